# baseline (speedup 1.0000x reference)
import numpy as np
import jax
import jax.numpy as jnp
from jax.experimental import pallas as pl
from jax.experimental.pallas import tpu as pltpu

_PRE_N = 2000
_POST_N = 1000
_THRESH = 0.7
_C = 5
_LANES = 128
_CAND_ROWS = 16


def _make_body(n_valid, rows):
    flat_big = np.int32(1 << 30)
    clipv = np.float32(np.log(1000.0 / 16.0))

    rm_rows = (rows + _LANES - 1) // _LANES

    def body(obj_ref, breg_ref, anch_ref, out_ref,
             score_ref, src_ref, cand_ref, sup_ref, rowmax_ref,
             count_ref, kidx_ref):
        ri = jax.lax.broadcasted_iota(jnp.int32, (rows, _LANES), 0)
        ci = jax.lax.broadcasted_iota(jnp.int32, (rows, _LANES), 1)
        flat = ri * _LANES + ci
        sig = jax.nn.sigmoid(obj_ref[0])
        score_ref[...] = jnp.where(flat < n_valid, sig, -1.0)

        dx = breg_ref[0, 0]
        dy = breg_ref[0, 1]
        dw = jnp.clip(breg_ref[0, 2], -clipv, clipv)
        dh = jnp.clip(breg_ref[0, 3], -clipv, clipv)
        dt = breg_ref[0, 4]
        xa = anch_ref[0, 0]
        ya = anch_ref[0, 1]
        wa = anch_ref[0, 2]
        ha = anch_ref[0, 3]
        ta = anch_ref[0, 4]
        src_ref[0] = dx * wa + xa
        src_ref[1] = dy * ha + ya
        src_ref[2] = wa * jnp.exp(dw)
        src_ref[3] = ha * jnp.exp(dh)
        src_ref[4] = dt * np.float32(180.0 / np.pi) + ta

        cflat = (jax.lax.broadcasted_iota(jnp.int32, (_CAND_ROWS, _LANES), 0)
                 * _LANES
                 + jax.lax.broadcasted_iota(jnp.int32, (_CAND_ROWS, _LANES), 1))
        sup_ref[...] = jnp.where(cflat < _PRE_N, 0.0, 1.0)
        cand_ref[...] = jnp.zeros((11, _CAND_ROWS, _LANES), jnp.float32)
        count_ref[0] = 0

        liota = jax.lax.broadcasted_iota(jnp.int32, (1, _LANES), 1)

        def lane_get(row, c):
            return jnp.sum(jnp.where(liota == c, row, 0.0))

        rowmax_ref[...] = jnp.full((rm_rows, _LANES), -1.0, jnp.float32)

        def rm_body(r, carry):
            mx = jnp.max(score_ref[pl.ds(r, 1), :])
            rr = r // _LANES
            rc = r - rr * _LANES
            row = rowmax_ref[pl.ds(rr, 1), :]
            rowmax_ref[pl.ds(rr, 1), :] = jnp.where(liota == rc, mx, row)
            return carry

        jax.lax.fori_loop(0, rows, rm_body, 0)

        rm_flat = (jax.lax.broadcasted_iota(jnp.int32, (rm_rows, _LANES), 0)
                   * _LANES
                   + jax.lax.broadcasted_iota(jnp.int32, (rm_rows, _LANES), 1))

        def sel_body(i, carry):
            rm = rowmax_ref[...]
            m = jnp.max(rm)
            r = jnp.min(jnp.where(rm == m, rm_flat, flat_big))
            srow = score_ref[pl.ds(r, 1), :]
            c = jnp.min(jnp.where(srow == m, liota, flat_big))
            zapped = jnp.where(liota == c, -1.0, srow)
            score_ref[pl.ds(r, 1), :] = zapped
            rr = r // _LANES
            rc = r - rr * _LANES
            rmrow = rowmax_ref[pl.ds(rr, 1), :]
            rowmax_ref[pl.ds(rr, 1), :] = jnp.where(
                liota == rc, jnp.max(zapped), rmrow)
            cr = i // _LANES
            cc = i - cr * _LANES
            for k in range(5):
                v = lane_get(src_ref[k, pl.ds(r, 1), :], c)
                crow = cand_ref[k, pl.ds(cr, 1), :]
                cand_ref[k, pl.ds(cr, 1), :] = jnp.where(
                    liota == cc, v, crow)
            crow = cand_ref[5, pl.ds(cr, 1), :]
            cand_ref[5, pl.ds(cr, 1), :] = jnp.where(liota == cc, m, crow)
            return carry

        jax.lax.fori_loop(0, _PRE_N, sel_body, 0)

        w_c = cand_ref[2]
        h_c = cand_ref[3]
        rad = cand_ref[4] * np.float32(np.pi / 180.0)
        cs = jnp.abs(jnp.cos(rad))
        sn = jnp.abs(jnp.sin(rad))
        bw = w_c * cs + h_c * sn
        bh = w_c * sn + h_c * cs
        x1 = cand_ref[0] - bw / 2
        y1 = cand_ref[1] - bh / 2
        x2 = cand_ref[0] + bw / 2
        y2 = cand_ref[1] + bh / 2
        cand_ref[6] = x1
        cand_ref[7] = y1
        cand_ref[8] = x2
        cand_ref[9] = y2
        cand_ref[10] = (x2 - x1) * (y2 - y1)

        def nms_body(i, carry):
            r = i // _LANES
            c = i - r * _LANES
            supv = lane_get(sup_ref[pl.ds(r, 1), :], c)
            cnt = count_ref[0]
            keep = jnp.logical_and(supv == 0.0, cnt < _POST_N)

            @pl.when(keep)
            def _():
                kidx_ref[cnt] = i
                count_ref[0] = cnt + 1
                x1i = lane_get(cand_ref[6, pl.ds(r, 1), :], c)
                y1i = lane_get(cand_ref[7, pl.ds(r, 1), :], c)
                x2i = lane_get(cand_ref[8, pl.ds(r, 1), :], c)
                y2i = lane_get(cand_ref[9, pl.ds(r, 1), :], c)
                ai = lane_get(cand_ref[10, pl.ds(r, 1), :], c)
                xx1 = jnp.maximum(cand_ref[6], x1i)
                yy1 = jnp.maximum(cand_ref[7], y1i)
                xx2 = jnp.minimum(cand_ref[8], x2i)
                yy2 = jnp.minimum(cand_ref[9], y2i)
                iw = jnp.maximum(xx2 - xx1, 0.0)
                ih = jnp.maximum(yy2 - yy1, 0.0)
                inter = iw * ih
                iou = inter / (ai + cand_ref[10] - inter + 1e-9)
                sup_ref[...] = jnp.maximum(
                    sup_ref[...], jnp.where(iou > _THRESH, 1.0, 0.0))

            return carry

        jax.lax.fori_loop(0, _PRE_N, nms_body, 0)

        cnt = count_ref[0]
        lastslot = kidx_ref[cnt - 1]

        def pad_body(j, carry):
            @pl.when(j >= cnt)
            def _():
                kidx_ref[j] = lastslot
            return carry

        jax.lax.fori_loop(0, _POST_N, pad_body, 0)

        oiota = jax.lax.broadcasted_iota(jnp.int32, (1, 6), 1)

        def out_body(j, carry):
            slot = kidx_ref[j]
            r = slot // _LANES
            c = slot - r * _LANES
            row = jnp.zeros((1, 6), jnp.float32)
            for k in range(6):
                v = lane_get(cand_ref[k, pl.ds(r, 1), :], c)
                row = jnp.where(oiota == k, v, row)
            out_ref[0, pl.ds(j, 1), :] = row
            return carry

        jax.lax.fori_loop(0, _POST_N, out_body, 0)

    return body


def kernel(objectness, box_regression, anchors_rrects):
    N, A, H, W = objectness.shape
    nA = A * H * W
    rows = ((nA + _LANES - 1) // _LANES + 7) // 8 * 8
    padded = rows * _LANES
    pad = padded - nA

    obj = objectness.reshape(N, A, 1, H, W).transpose(0, 3, 4, 1, 2)
    obj = obj.reshape(N, nA)
    breg = box_regression.reshape(N, A, _C, H, W).transpose(0, 3, 4, 1, 2)
    breg = breg.reshape(N, nA, _C)

    obj_p = jnp.pad(obj, ((0, 0), (0, pad))).reshape(N, rows, _LANES)
    breg_t = jnp.pad(breg.transpose(0, 2, 1),
                     ((0, 0), (0, 0), (0, pad))).reshape(N, _C, rows, _LANES)
    anch_t = jnp.pad(anchors_rrects.transpose(0, 2, 1),
                     ((0, 0), (0, 0), (0, pad))).reshape(N, _C, rows, _LANES)

    out = pl.pallas_call(
        _make_body(nA, rows),
        grid=(N,),
        in_specs=[
            pl.BlockSpec((1, rows, _LANES), lambda n: (n, 0, 0)),
            pl.BlockSpec((1, _C, rows, _LANES), lambda n: (n, 0, 0, 0)),
            pl.BlockSpec((1, _C, rows, _LANES), lambda n: (n, 0, 0, 0)),
        ],
        out_specs=pl.BlockSpec((1, _POST_N, 6), lambda n: (n, 0, 0)),
        out_shape=jax.ShapeDtypeStruct((N, _POST_N, 6), jnp.float32),
        scratch_shapes=[
            pltpu.VMEM((rows, _LANES), jnp.float32),
            pltpu.VMEM((5, rows, _LANES), jnp.float32),
            pltpu.VMEM((11, _CAND_ROWS, _LANES), jnp.float32),
            pltpu.VMEM((_CAND_ROWS, _LANES), jnp.float32),
            pltpu.VMEM(((rows + _LANES - 1) // _LANES, _LANES), jnp.float32),
            pltpu.SMEM((1,), jnp.int32),
            pltpu.SMEM((_POST_N,), jnp.int32),
        ],
    )(obj_p, breg_t, anch_t)
    return out


# speedup vs baseline: 1.2425x; 1.2425x over previous
import numpy as np
import jax
import jax.numpy as jnp
from jax.experimental import pallas as pl
from jax.experimental.pallas import tpu as pltpu

_PRE_N = 2000
_POST_N = 1000
_THRESH = 0.7
_C = 5
_LANES = 128
_CAND_ROWS = 16


def _make_body(n_valid, rows):
    flat_big = np.int32(1 << 30)
    clipv = np.float32(np.log(1000.0 / 16.0))

    def body(obj_ref, breg_ref, anch_ref, out_ref,
             score_ref, src_ref, cand_ref, sup_ref,
             count_ref, kidx_ref):
        ri = jax.lax.broadcasted_iota(jnp.int32, (rows, _LANES), 0)
        ci = jax.lax.broadcasted_iota(jnp.int32, (rows, _LANES), 1)
        flat = ri * _LANES + ci
        sig = jax.nn.sigmoid(obj_ref[0])
        score_ref[...] = jnp.where(flat < n_valid, sig, -1.0)

        dx = breg_ref[0, 0]
        dy = breg_ref[0, 1]
        dw = jnp.clip(breg_ref[0, 2], -clipv, clipv)
        dh = jnp.clip(breg_ref[0, 3], -clipv, clipv)
        dt = breg_ref[0, 4]
        xa = anch_ref[0, 0]
        ya = anch_ref[0, 1]
        wa = anch_ref[0, 2]
        ha = anch_ref[0, 3]
        ta = anch_ref[0, 4]
        src_ref[0] = dx * wa + xa
        src_ref[1] = dy * ha + ya
        src_ref[2] = wa * jnp.exp(dw)
        src_ref[3] = ha * jnp.exp(dh)
        src_ref[4] = dt * np.float32(180.0 / np.pi) + ta

        cflat = (jax.lax.broadcasted_iota(jnp.int32, (_CAND_ROWS, _LANES), 0)
                 * _LANES
                 + jax.lax.broadcasted_iota(jnp.int32, (_CAND_ROWS, _LANES), 1))
        sup_ref[...] = jnp.where(cflat < _PRE_N, 0.0, 1.0)
        cand_ref[...] = jnp.zeros((11, _CAND_ROWS, _LANES), jnp.float32)
        count_ref[0] = 0

        liota = jax.lax.broadcasted_iota(jnp.int32, (1, _LANES), 1)

        def lane_get(row, c):
            return jnp.sum(jnp.where(liota == c, row, 0.0))

        def sel_body(i, carry):
            s = score_ref[...]
            m = jnp.max(s)
            idx = jnp.min(jnp.where(s == m, flat, flat_big))
            r = idx // _LANES
            c = idx - r * _LANES
            srow = score_ref[pl.ds(r, 1), :]
            score_ref[pl.ds(r, 1), :] = jnp.where(liota == c, -1.0, srow)
            cr = i // _LANES
            cc = i - cr * _LANES
            for k in range(5):
                v = lane_get(src_ref[k, pl.ds(r, 1), :], c)
                crow = cand_ref[k, pl.ds(cr, 1), :]
                cand_ref[k, pl.ds(cr, 1), :] = jnp.where(
                    liota == cc, v, crow)
            crow = cand_ref[5, pl.ds(cr, 1), :]
            cand_ref[5, pl.ds(cr, 1), :] = jnp.where(liota == cc, m, crow)
            return carry

        jax.lax.fori_loop(0, _PRE_N, sel_body, 0)

        w_c = cand_ref[2]
        h_c = cand_ref[3]
        rad = cand_ref[4] * np.float32(np.pi / 180.0)
        cs = jnp.abs(jnp.cos(rad))
        sn = jnp.abs(jnp.sin(rad))
        bw = w_c * cs + h_c * sn
        bh = w_c * sn + h_c * cs
        x1 = cand_ref[0] - bw / 2
        y1 = cand_ref[1] - bh / 2
        x2 = cand_ref[0] + bw / 2
        y2 = cand_ref[1] + bh / 2
        cand_ref[6] = x1
        cand_ref[7] = y1
        cand_ref[8] = x2
        cand_ref[9] = y2
        cand_ref[10] = (x2 - x1) * (y2 - y1)

        def nms_body(i, carry):
            r = i // _LANES
            c = i - r * _LANES
            supv = lane_get(sup_ref[pl.ds(r, 1), :], c)
            cnt = count_ref[0]
            keep = jnp.logical_and(supv == 0.0, cnt < _POST_N)

            @pl.when(keep)
            def _():
                kidx_ref[cnt] = i
                count_ref[0] = cnt + 1
                x1i = lane_get(cand_ref[6, pl.ds(r, 1), :], c)
                y1i = lane_get(cand_ref[7, pl.ds(r, 1), :], c)
                x2i = lane_get(cand_ref[8, pl.ds(r, 1), :], c)
                y2i = lane_get(cand_ref[9, pl.ds(r, 1), :], c)
                ai = lane_get(cand_ref[10, pl.ds(r, 1), :], c)
                xx1 = jnp.maximum(cand_ref[6], x1i)
                yy1 = jnp.maximum(cand_ref[7], y1i)
                xx2 = jnp.minimum(cand_ref[8], x2i)
                yy2 = jnp.minimum(cand_ref[9], y2i)
                iw = jnp.maximum(xx2 - xx1, 0.0)
                ih = jnp.maximum(yy2 - yy1, 0.0)
                inter = iw * ih
                iou = inter / (ai + cand_ref[10] - inter + 1e-9)
                sup_ref[...] = jnp.maximum(
                    sup_ref[...], jnp.where(iou > _THRESH, 1.0, 0.0))

            return carry

        jax.lax.fori_loop(0, _PRE_N, nms_body, 0)

        cnt = count_ref[0]
        lastslot = kidx_ref[cnt - 1]

        def pad_body(j, carry):
            @pl.when(j >= cnt)
            def _():
                kidx_ref[j] = lastslot
            return carry

        jax.lax.fori_loop(0, _POST_N, pad_body, 0)

        oiota = jax.lax.broadcasted_iota(jnp.int32, (1, 6), 1)

        def out_body(j, carry):
            slot = kidx_ref[j]
            r = slot // _LANES
            c = slot - r * _LANES
            row = jnp.zeros((1, 6), jnp.float32)
            for k in range(6):
                v = lane_get(cand_ref[k, pl.ds(r, 1), :], c)
                row = jnp.where(oiota == k, v, row)
            out_ref[0, pl.ds(j, 1), :] = row
            return carry

        jax.lax.fori_loop(0, _POST_N, out_body, 0)

    return body


def kernel(objectness, box_regression, anchors_rrects):
    N, A, H, W = objectness.shape
    nA = A * H * W
    rows = ((nA + _LANES - 1) // _LANES + 7) // 8 * 8
    padded = rows * _LANES
    pad = padded - nA

    obj = objectness.reshape(N, A, 1, H, W).transpose(0, 3, 4, 1, 2)
    obj = obj.reshape(N, nA)
    breg = box_regression.reshape(N, A, _C, H, W).transpose(0, 3, 4, 1, 2)
    breg = breg.reshape(N, nA, _C)

    obj_p = jnp.pad(obj, ((0, 0), (0, pad))).reshape(N, rows, _LANES)
    breg_t = jnp.pad(breg.transpose(0, 2, 1),
                     ((0, 0), (0, 0), (0, pad))).reshape(N, _C, rows, _LANES)
    anch_t = jnp.pad(anchors_rrects.transpose(0, 2, 1),
                     ((0, 0), (0, 0), (0, pad))).reshape(N, _C, rows, _LANES)

    out = pl.pallas_call(
        _make_body(nA, rows),
        grid=(N,),
        in_specs=[
            pl.BlockSpec((1, rows, _LANES), lambda n: (n, 0, 0)),
            pl.BlockSpec((1, _C, rows, _LANES), lambda n: (n, 0, 0, 0)),
            pl.BlockSpec((1, _C, rows, _LANES), lambda n: (n, 0, 0, 0)),
        ],
        out_specs=pl.BlockSpec((1, _POST_N, 6), lambda n: (n, 0, 0)),
        out_shape=jax.ShapeDtypeStruct((N, _POST_N, 6), jnp.float32),
        scratch_shapes=[
            pltpu.VMEM((rows, _LANES), jnp.float32),
            pltpu.VMEM((5, rows, _LANES), jnp.float32),
            pltpu.VMEM((11, _CAND_ROWS, _LANES), jnp.float32),
            pltpu.VMEM((_CAND_ROWS, _LANES), jnp.float32),
            pltpu.SMEM((1,), jnp.int32),
            pltpu.SMEM((_POST_N,), jnp.int32),
        ],
    )(obj_p, breg_t, anch_t)
    return out


# speedup vs baseline: 3.0697x; 2.4707x over previous
import numpy as np
import jax
import jax.numpy as jnp
from jax.experimental import pallas as pl
from jax.experimental.pallas import tpu as pltpu

_PRE_N = 2000
_POST_N = 1000
_THRESH = 0.7
_C = 5
_LANES = 128


def _make_body(n_valid, rows):
    flat_big = np.int32(1 << 30)
    clipv = np.float32(np.log(1000.0 / 16.0))

    def body(obj_ref, breg_ref, anch_ref, out_ref,
             score_ref, src_ref, lastrow_ref):
        ri = jax.lax.broadcasted_iota(jnp.int32, (rows, _LANES), 0)
        ci = jax.lax.broadcasted_iota(jnp.int32, (rows, _LANES), 1)
        flat = ri * _LANES + ci
        valid = flat < n_valid
        sig = jax.nn.sigmoid(obj_ref[0])
        score_ref[...] = jnp.where(valid, sig, -1.0)

        dx = breg_ref[0, 0]
        dy = breg_ref[0, 1]
        dw = jnp.clip(breg_ref[0, 2], -clipv, clipv)
        dh = jnp.clip(breg_ref[0, 3], -clipv, clipv)
        dt = breg_ref[0, 4]
        xa = anch_ref[0, 0]
        ya = anch_ref[0, 1]
        wa = anch_ref[0, 2]
        ha = anch_ref[0, 3]
        ta = anch_ref[0, 4]
        xc = dx * wa + xa
        yc = dy * ha + ya
        w = wa * jnp.exp(dw)
        h = ha * jnp.exp(dh)
        th = dt * np.float32(180.0 / np.pi) + ta
        rad = th * np.float32(np.pi / 180.0)
        cs = jnp.abs(jnp.cos(rad))
        sn = jnp.abs(jnp.sin(rad))
        bw = w * cs + h * sn
        bh = w * sn + h * cs
        x1 = xc - bw / 2
        y1 = yc - bh / 2
        x2 = xc + bw / 2
        y2 = yc + bh / 2
        src_ref[0] = xc
        src_ref[1] = yc
        src_ref[2] = w
        src_ref[3] = h
        src_ref[4] = th
        src_ref[5] = x1
        src_ref[6] = y1
        src_ref[7] = x2
        src_ref[8] = y2
        src_ref[9] = (x2 - x1) * (y2 - y1)

        def tstep(j, tbits):
            cand = tbits | (jnp.int32(1) << (30 - j))
            sbits = jax.lax.bitcast_convert_type(score_ref[...], jnp.int32)
            cnt = jnp.sum(jnp.where(sbits >= cand, 1, 0))
            return jnp.where(cnt >= _PRE_N, cand, tbits)

        tbits = jax.lax.fori_loop(0, 31, tstep, jnp.int32(0))
        sbits = jax.lax.bitcast_convert_type(score_ref[...], jnp.int32)
        n_above = jnp.sum(jnp.where(sbits > tbits, 1, 0))
        k_ties = _PRE_N - n_above
        eq = sbits == tbits

        def xstep(j, x):
            candx = x + (jnp.int32(1) << (15 - j))
            cnt = jnp.sum(jnp.where(jnp.logical_and(eq, flat < candx), 1, 0))
            return jnp.where(cnt < k_ties, candx, x)

        xcut = jax.lax.fori_loop(0, 16, xstep, jnp.int32(0))
        active = jnp.logical_or(sbits > tbits,
                                jnp.logical_and(eq, flat <= xcut))
        score_ref[...] = jnp.where(active, score_ref[...], -1.0)

        liota = jax.lax.broadcasted_iota(jnp.int32, (1, _LANES), 1)
        oiota = jax.lax.broadcasted_iota(jnp.int32, (1, 6), 1)

        def lane_get(row, c):
            return jnp.sum(jnp.where(liota == c, row, 0.0))

        m0 = jnp.max(score_ref[...])

        def cond_fn(carry):
            cnt, m = carry
            return jnp.logical_and(cnt < _POST_N, m > -0.5)

        def body_fn(carry):
            cnt, m = carry
            s = score_ref[...]
            idx = jnp.min(jnp.where(s == m, flat, flat_big))
            r = idx // _LANES
            c = idx - r * _LANES
            g = [lane_get(src_ref[k, pl.ds(r, 1), :], c) for k in range(10)]
            row = jnp.zeros((1, 6), jnp.float32)
            vals6 = [g[0], g[1], g[2], g[3], g[4], m]
            for k in range(6):
                row = jnp.where(oiota == k, vals6[k], row)
                lastrow_ref[k] = vals6[k]
            out_ref[0, pl.ds(cnt, 1), :] = row
            x1i, y1i, x2i, y2i, ai = g[5], g[6], g[7], g[8], g[9]
            xx1 = jnp.maximum(src_ref[5], x1i)
            yy1 = jnp.maximum(src_ref[6], y1i)
            xx2 = jnp.minimum(src_ref[7], x2i)
            yy2 = jnp.minimum(src_ref[8], y2i)
            iw = jnp.maximum(xx2 - xx1, 0.0)
            ih = jnp.maximum(yy2 - yy1, 0.0)
            inter = iw * ih
            iou = inter / (ai + src_ref[9] - inter + 1e-9)
            news = jnp.where(iou > _THRESH, -1.0, s)
            score_ref[...] = news
            return cnt + 1, jnp.max(news)

        cnt, _ = jax.lax.while_loop(cond_fn, body_fn, (jnp.int32(0), m0))

        prow = jnp.zeros((1, 6), jnp.float32)
        for k in range(6):
            prow = jnp.where(oiota == k, lastrow_ref[k], prow)

        def pad_body(j, carry):
            @pl.when(j >= cnt)
            def _():
                out_ref[0, pl.ds(j, 1), :] = prow
            return carry

        jax.lax.fori_loop(0, _POST_N, pad_body, 0)

    return body


def kernel(objectness, box_regression, anchors_rrects):
    N, A, H, W = objectness.shape
    nA = A * H * W
    rows = ((nA + _LANES - 1) // _LANES + 7) // 8 * 8
    padded = rows * _LANES
    pad = padded - nA

    obj = objectness.reshape(N, A, 1, H, W).transpose(0, 3, 4, 1, 2)
    obj = obj.reshape(N, nA)
    breg = box_regression.reshape(N, A, _C, H, W).transpose(0, 3, 4, 1, 2)
    breg = breg.reshape(N, nA, _C)

    obj_p = jnp.pad(obj, ((0, 0), (0, pad))).reshape(N, rows, _LANES)
    breg_t = jnp.pad(breg.transpose(0, 2, 1),
                     ((0, 0), (0, 0), (0, pad))).reshape(N, _C, rows, _LANES)
    anch_t = jnp.pad(anchors_rrects.transpose(0, 2, 1),
                     ((0, 0), (0, 0), (0, pad))).reshape(N, _C, rows, _LANES)

    out = pl.pallas_call(
        _make_body(nA, rows),
        grid=(N,),
        in_specs=[
            pl.BlockSpec((1, rows, _LANES), lambda n: (n, 0, 0)),
            pl.BlockSpec((1, _C, rows, _LANES), lambda n: (n, 0, 0, 0)),
            pl.BlockSpec((1, _C, rows, _LANES), lambda n: (n, 0, 0, 0)),
        ],
        out_specs=pl.BlockSpec((1, _POST_N, 6), lambda n: (n, 0, 0)),
        out_shape=jax.ShapeDtypeStruct((N, _POST_N, 6), jnp.float32),
        scratch_shapes=[
            pltpu.VMEM((rows, _LANES), jnp.float32),
            pltpu.VMEM((10, rows, _LANES), jnp.float32),
            pltpu.SMEM((6,), jnp.float32),
        ],
    )(obj_p, breg_t, anch_t)
    return out


# speedup vs baseline: 3.2646x; 1.0635x over previous
import numpy as np
import jax
import jax.numpy as jnp
from jax.experimental import pallas as pl
from jax.experimental.pallas import tpu as pltpu

_PRE_N = 2000
_POST_N = 1000
_THRESH = 0.7
_C = 5
_LANES = 128


def _make_body(n_valid, rows, n_img):
    flat_big = np.int32(1 << 30)
    clipv = np.float32(np.log(1000.0 / 16.0))

    def body(obj_ref, breg_ref, anch_ref, out_ref,
             score_ref, src_ref, lastrow_ref):
        ri = jax.lax.broadcasted_iota(jnp.int32, (rows, _LANES), 0)
        ci = jax.lax.broadcasted_iota(jnp.int32, (rows, _LANES), 1)
        flat = ri * _LANES + ci
        valid = flat < n_valid

        for n in range(n_img):
            sig = jax.nn.sigmoid(obj_ref[n])
            score_ref[n] = jnp.where(valid, sig, -1.0)

            dx = breg_ref[n, 0]
            dy = breg_ref[n, 1]
            dw = jnp.clip(breg_ref[n, 2], -clipv, clipv)
            dh = jnp.clip(breg_ref[n, 3], -clipv, clipv)
            dt = breg_ref[n, 4]
            xa = anch_ref[n, 0]
            ya = anch_ref[n, 1]
            wa = anch_ref[n, 2]
            ha = anch_ref[n, 3]
            ta = anch_ref[n, 4]
            xc = dx * wa + xa
            yc = dy * ha + ya
            w = wa * jnp.exp(dw)
            h = ha * jnp.exp(dh)
            th = dt * np.float32(180.0 / np.pi) + ta
            rad = th * np.float32(np.pi / 180.0)
            cs = jnp.abs(jnp.cos(rad))
            sn = jnp.abs(jnp.sin(rad))
            bw = w * cs + h * sn
            bh = w * sn + h * cs
            x1 = xc - bw / 2
            y1 = yc - bh / 2
            x2 = xc + bw / 2
            y2 = yc + bh / 2
            src_ref[n, 0] = xc
            src_ref[n, 1] = yc
            src_ref[n, 2] = w
            src_ref[n, 3] = h
            src_ref[n, 4] = th
            src_ref[n, 5] = x1
            src_ref[n, 6] = y1
            src_ref[n, 7] = x2
            src_ref[n, 8] = y2
            src_ref[n, 9] = (x2 - x1) * (y2 - y1)

            def tstep(j, tbits):
                cand = tbits | (jnp.int32(1) << (30 - j))
                sbits = jax.lax.bitcast_convert_type(score_ref[n], jnp.int32)
                cnt = jnp.sum(jnp.where(sbits >= cand, 1, 0))
                return jnp.where(cnt >= _PRE_N, cand, tbits)

            tbits = jax.lax.fori_loop(0, 31, tstep, jnp.int32(0))
            sbits = jax.lax.bitcast_convert_type(score_ref[n], jnp.int32)
            n_above = jnp.sum(jnp.where(sbits > tbits, 1, 0))
            k_ties = _PRE_N - n_above
            eq = sbits == tbits

            def xstep(j, x):
                candx = x + (jnp.int32(1) << (15 - j))
                cnt = jnp.sum(
                    jnp.where(jnp.logical_and(eq, flat < candx), 1, 0))
                return jnp.where(cnt < k_ties, candx, x)

            xcut = jax.lax.fori_loop(0, 16, xstep, jnp.int32(0))
            act = jnp.logical_or(sbits > tbits,
                                 jnp.logical_and(eq, flat <= xcut))
            score_ref[n] = jnp.where(act, score_ref[n], -1.0)

        liota = jax.lax.broadcasted_iota(jnp.int32, (1, _LANES), 1)
        oiota = jax.lax.broadcasted_iota(jnp.int32, (1, 6), 1)

        def lane_get(row, c):
            return jnp.sum(jnp.where(liota == c, row, 0.0))

        m0s = [jnp.max(score_ref[n]) for n in range(n_img)]
        cnt0s = [jnp.int32(0)] * n_img

        def cond_fn(carry):
            cnts, ms = carry
            alive = [jnp.logical_and(cnts[n] < _POST_N, ms[n] > -0.5)
                     for n in range(n_img)]
            r = alive[0]
            for a in alive[1:]:
                r = jnp.logical_or(r, a)
            return r

        def body_fn(carry):
            cnts, ms = carry
            new_cnts = []
            new_ms = []
            for n in range(n_img):
                cnt, m = cnts[n], ms[n]
                active = jnp.logical_and(cnt < _POST_N, m > -0.5)
                s = score_ref[n]
                idx = jnp.min(jnp.where(s == m, flat, flat_big))
                r = idx // _LANES
                c = idx - r * _LANES
                g = [lane_get(src_ref[n, k, pl.ds(r, 1), :], c)
                     for k in range(10)]
                row = jnp.zeros((1, 6), jnp.float32)
                vals6 = [g[0], g[1], g[2], g[3], g[4], m]
                for k in range(6):
                    row = jnp.where(oiota == k, vals6[k], row)

                @pl.when(active)
                def _(n=n, row=row, vals6=vals6, cnt=cnt):
                    out_ref[n, pl.ds(cnt, 1), :] = row
                    for k in range(6):
                        lastrow_ref[n, k] = vals6[k]

                x1i, y1i, x2i, y2i, ai = g[5], g[6], g[7], g[8], g[9]
                xx1 = jnp.maximum(src_ref[n, 5], x1i)
                yy1 = jnp.maximum(src_ref[n, 6], y1i)
                xx2 = jnp.minimum(src_ref[n, 7], x2i)
                yy2 = jnp.minimum(src_ref[n, 8], y2i)
                iw = jnp.maximum(xx2 - xx1, 0.0)
                ih = jnp.maximum(yy2 - yy1, 0.0)
                inter = iw * ih
                iou = inter / (ai + src_ref[n, 9] - inter + 1e-9)
                news = jnp.where(iou > _THRESH, -1.0, s)

                @pl.when(active)
                def _(n=n, news=news):
                    score_ref[n] = news

                new_cnts.append(jnp.where(active, cnt + 1, cnt))
                new_ms.append(jnp.where(active, jnp.max(news), m))
            return tuple(new_cnts), tuple(new_ms)

        final_cnts, _ = jax.lax.while_loop(
            cond_fn, body_fn, (tuple(cnt0s), tuple(m0s)))

        prows = []
        for n in range(n_img):
            prow = jnp.zeros((1, 6), jnp.float32)
            for k in range(6):
                prow = jnp.where(oiota == k, lastrow_ref[n, k], prow)
            prows.append(prow)

        def pad_body(j, carry):
            for n in range(n_img):
                @pl.when(j >= final_cnts[n])
                def _(n=n):
                    out_ref[n, pl.ds(j, 1), :] = prows[n]
            return carry

        jax.lax.fori_loop(0, _POST_N, pad_body, 0)

    return body


def kernel(objectness, box_regression, anchors_rrects):
    N, A, H, W = objectness.shape
    nA = A * H * W
    rows = ((nA + _LANES - 1) // _LANES + 7) // 8 * 8
    padded = rows * _LANES
    pad = padded - nA

    obj = objectness.reshape(N, A, 1, H, W).transpose(0, 3, 4, 1, 2)
    obj = obj.reshape(N, nA)
    breg = box_regression.reshape(N, A, _C, H, W).transpose(0, 3, 4, 1, 2)
    breg = breg.reshape(N, nA, _C)

    obj_p = jnp.pad(obj, ((0, 0), (0, pad))).reshape(N, rows, _LANES)
    breg_t = jnp.pad(breg.transpose(0, 2, 1),
                     ((0, 0), (0, 0), (0, pad))).reshape(N, _C, rows, _LANES)
    anch_t = jnp.pad(anchors_rrects.transpose(0, 2, 1),
                     ((0, 0), (0, 0), (0, pad))).reshape(N, _C, rows, _LANES)

    out = pl.pallas_call(
        _make_body(nA, rows, N),
        grid=(1,),
        in_specs=[
            pl.BlockSpec((N, rows, _LANES), lambda i: (0, 0, 0)),
            pl.BlockSpec((N, _C, rows, _LANES), lambda i: (0, 0, 0, 0)),
            pl.BlockSpec((N, _C, rows, _LANES), lambda i: (0, 0, 0, 0)),
        ],
        out_specs=pl.BlockSpec((N, _POST_N, 6), lambda i: (0, 0, 0)),
        out_shape=jax.ShapeDtypeStruct((N, _POST_N, 6), jnp.float32),
        scratch_shapes=[
            pltpu.VMEM((N, rows, _LANES), jnp.float32),
            pltpu.VMEM((N, 10, rows, _LANES), jnp.float32),
            pltpu.SMEM((N, 6), jnp.float32),
        ],
    )(obj_p, breg_t, anch_t)
    return out


# speedup vs baseline: 5.7759x; 1.7692x over previous
import numpy as np
import jax
import jax.numpy as jnp
from jax.experimental import pallas as pl
from jax.experimental.pallas import tpu as pltpu

_PRE_N = 2000
_POST_N = 1000
_THRESH = 0.7
_C = 5
_LANES = 128


def _make_body(n_valid, rows, n_img):
    flat_big = np.int32(1 << 30)
    clipv = np.float32(np.log(1000.0 / 16.0))

    def body(obj_ref, breg_ref, anch_ref, out_ref,
             score_ref, src_ref):
        ri = jax.lax.broadcasted_iota(jnp.int32, (rows, _LANES), 0)
        ci = jax.lax.broadcasted_iota(jnp.int32, (rows, _LANES), 1)
        flat = ri * _LANES + ci
        valid = flat < n_valid

        for n in range(n_img):
            sig = jax.nn.sigmoid(obj_ref[n])
            score_ref[n] = jnp.where(valid, sig, -1.0)

            dx = breg_ref[n, 0]
            dy = breg_ref[n, 1]
            dw = jnp.clip(breg_ref[n, 2], -clipv, clipv)
            dh = jnp.clip(breg_ref[n, 3], -clipv, clipv)
            dt = breg_ref[n, 4]
            xa = anch_ref[n, 0]
            ya = anch_ref[n, 1]
            wa = anch_ref[n, 2]
            ha = anch_ref[n, 3]
            ta = anch_ref[n, 4]
            xc = dx * wa + xa
            yc = dy * ha + ya
            w = wa * jnp.exp(dw)
            h = ha * jnp.exp(dh)
            th = dt * np.float32(180.0 / np.pi) + ta
            rad = th * np.float32(np.pi / 180.0)
            cs = jnp.abs(jnp.cos(rad))
            sn = jnp.abs(jnp.sin(rad))
            bw = w * cs + h * sn
            bh = w * sn + h * cs
            x1 = xc - bw / 2
            y1 = yc - bh / 2
            x2 = xc + bw / 2
            y2 = yc + bh / 2
            src_ref[n, 0] = xc
            src_ref[n, 1] = yc
            src_ref[n, 2] = w
            src_ref[n, 3] = h
            src_ref[n, 4] = th
            src_ref[n, 5] = x1
            src_ref[n, 6] = y1
            src_ref[n, 7] = x2
            src_ref[n, 8] = y2
            src_ref[n, 9] = (x2 - x1) * (y2 - y1)
            src_ref[n, 10] = bw
            src_ref[n, 11] = bh

            def tstep(j, tbits):
                cand = tbits | (jnp.int32(1) << (30 - j))
                sbits = jax.lax.bitcast_convert_type(score_ref[n], jnp.int32)
                cnt = jnp.sum(jnp.where(sbits >= cand, 1, 0))
                return jnp.where(cnt >= _PRE_N, cand, tbits)

            tbits = jax.lax.fori_loop(0, 31, tstep, jnp.int32(0))
            sbits = jax.lax.bitcast_convert_type(score_ref[n], jnp.int32)
            n_above = jnp.sum(jnp.where(sbits > tbits, 1, 0))
            k_ties = _PRE_N - n_above
            eq = sbits == tbits

            def xstep(j, x):
                candx = x + (jnp.int32(1) << (15 - j))
                cnt = jnp.sum(
                    jnp.where(jnp.logical_and(eq, flat < candx), 1, 0))
                return jnp.where(cnt < k_ties, candx, x)

            xcut = jax.lax.fori_loop(0, 16, xstep, jnp.int32(0))
            act = jnp.logical_or(sbits > tbits,
                                 jnp.logical_and(eq, flat <= xcut))
            score_ref[n] = jnp.where(act, score_ref[n], -1.0)

        oiota = jax.lax.broadcasted_iota(jnp.int32, (1, 6), 1)
        flat_f = flat.astype(jnp.float32)
        big_f = np.float32(1 << 30)

        def vmax11(x):
            return jnp.max(jnp.max(x, axis=0, keepdims=True),
                           axis=1, keepdims=True)

        def vmin11(x):
            return jnp.min(jnp.min(x, axis=0, keepdims=True),
                           axis=1, keepdims=True)

        def vsum11(x):
            return jnp.sum(jnp.sum(x, axis=0, keepdims=True),
                           axis=1, keepdims=True)

        init = []
        for n in range(n_img):
            init.append(vmax11(score_ref[n]))
            init.append(jnp.zeros((1, 6), jnp.float32))

        def body_fn(j, carry):
            new_carry = []
            for n in range(n_img):
                m = carry[2 * n]
                lastrow = carry[2 * n + 1]
                s = score_ref[n]
                hit = s == m
                idxv = vmin11(jnp.where(hit, flat_f, big_f))
                selm = jnp.where(jnp.logical_and(hit, flat_f == idxv),
                                 1.0, 0.0)
                g = [vsum11(src_ref[n, k] * selm)
                     for k in (0, 1, 2, 3, 4, 10, 11)]
                xci, yci, wi, hi, thi, bwi, bhi = g
                x1i = xci - bwi / 2
                y1i = yci - bhi / 2
                x2i = xci + bwi / 2
                y2i = yci + bhi / 2
                ai = (x2i - x1i) * (y2i - y1i)
                arow = jnp.zeros((1, 6), jnp.float32)
                vals6 = [xci, yci, wi, hi, thi, m]
                for k in range(6):
                    arow = jnp.where(oiota == k, vals6[k], arow)
                row = jnp.where(m > -0.5, arow, lastrow)
                out_ref[n, pl.ds(j, 1), :] = row
                xx1 = jnp.maximum(src_ref[n, 5], x1i)
                yy1 = jnp.maximum(src_ref[n, 6], y1i)
                xx2 = jnp.minimum(src_ref[n, 7], x2i)
                yy2 = jnp.minimum(src_ref[n, 8], y2i)
                iw = jnp.maximum(xx2 - xx1, 0.0)
                ih = jnp.maximum(yy2 - yy1, 0.0)
                inter = iw * ih
                iou = inter / (ai + src_ref[n, 9] - inter + 1e-9)
                news = jnp.where(iou > _THRESH, -1.0, s)
                score_ref[n] = news
                new_carry.append(vmax11(news))
                new_carry.append(row)
            return tuple(new_carry)

        jax.lax.fori_loop(0, _POST_N, body_fn, tuple(init))

    return body


def kernel(objectness, box_regression, anchors_rrects):
    N, A, H, W = objectness.shape
    nA = A * H * W
    rows = ((nA + _LANES - 1) // _LANES + 7) // 8 * 8
    padded = rows * _LANES
    pad = padded - nA

    obj = objectness.reshape(N, A, 1, H, W).transpose(0, 3, 4, 1, 2)
    obj = obj.reshape(N, nA)
    breg = box_regression.reshape(N, A, _C, H, W).transpose(0, 3, 4, 1, 2)
    breg = breg.reshape(N, nA, _C)

    obj_p = jnp.pad(obj, ((0, 0), (0, pad))).reshape(N, rows, _LANES)
    breg_t = jnp.pad(breg.transpose(0, 2, 1),
                     ((0, 0), (0, 0), (0, pad))).reshape(N, _C, rows, _LANES)
    anch_t = jnp.pad(anchors_rrects.transpose(0, 2, 1),
                     ((0, 0), (0, 0), (0, pad))).reshape(N, _C, rows, _LANES)

    out = pl.pallas_call(
        _make_body(nA, rows, N),
        grid=(1,),
        in_specs=[
            pl.BlockSpec((N, rows, _LANES), lambda i: (0, 0, 0)),
            pl.BlockSpec((N, _C, rows, _LANES), lambda i: (0, 0, 0, 0)),
            pl.BlockSpec((N, _C, rows, _LANES), lambda i: (0, 0, 0, 0)),
        ],
        out_specs=pl.BlockSpec((N, _POST_N, 6), lambda i: (0, 0, 0)),
        out_shape=jax.ShapeDtypeStruct((N, _POST_N, 6), jnp.float32),
        scratch_shapes=[
            pltpu.VMEM((N, rows, _LANES), jnp.float32),
            pltpu.VMEM((N, 12, rows, _LANES), jnp.float32),
        ],
    )(obj_p, breg_t, anch_t)
    return out


# speedup vs baseline: 5.8442x; 1.0118x over previous
import numpy as np
import jax
import jax.numpy as jnp
from jax.experimental import pallas as pl
from jax.experimental.pallas import tpu as pltpu

_PRE_N = 2000
_POST_N = 1000
_THRESH = 0.7
_C = 5
_LANES = 128


def _make_body(n_valid, rows, n_img):
    flat_big = np.int32(1 << 30)
    clipv = np.float32(np.log(1000.0 / 16.0))

    def body(obj_ref, breg_ref, anch_ref, out_ref,
             score_ref, src_ref):
        ri = jax.lax.broadcasted_iota(jnp.int32, (rows, _LANES), 0)
        ci = jax.lax.broadcasted_iota(jnp.int32, (rows, _LANES), 1)
        flat = ri * _LANES + ci
        valid = flat < n_valid

        for n in range(n_img):
            sig = jax.nn.sigmoid(obj_ref[n])
            score_ref[n] = jnp.where(valid, sig, -1.0)

            dx = breg_ref[n, 0]
            dy = breg_ref[n, 1]
            dw = jnp.clip(breg_ref[n, 2], -clipv, clipv)
            dh = jnp.clip(breg_ref[n, 3], -clipv, clipv)
            dt = breg_ref[n, 4]
            xa = anch_ref[n, 0]
            ya = anch_ref[n, 1]
            wa = anch_ref[n, 2]
            ha = anch_ref[n, 3]
            ta = anch_ref[n, 4]
            xc = dx * wa + xa
            yc = dy * ha + ya
            w = wa * jnp.exp(dw)
            h = ha * jnp.exp(dh)
            th = dt * np.float32(180.0 / np.pi) + ta
            rad = th * np.float32(np.pi / 180.0)
            cs = jnp.abs(jnp.cos(rad))
            sn = jnp.abs(jnp.sin(rad))
            bw = w * cs + h * sn
            bh = w * sn + h * cs
            x1 = xc - bw / 2
            y1 = yc - bh / 2
            x2 = xc + bw / 2
            y2 = yc + bh / 2
            src_ref[n, 0] = xc
            src_ref[n, 1] = yc
            src_ref[n, 2] = w
            src_ref[n, 3] = h
            src_ref[n, 4] = th
            src_ref[n, 5] = x1
            src_ref[n, 6] = y1
            src_ref[n, 7] = x2
            src_ref[n, 8] = y2
            src_ref[n, 9] = (x2 - x1) * (y2 - y1)
            src_ref[n, 10] = bw
            src_ref[n, 11] = bh

            def tstep(j, tbits):
                cand = tbits | (jnp.int32(1) << (30 - j))
                sbits = jax.lax.bitcast_convert_type(score_ref[n], jnp.int32)
                cnt = jnp.sum(jnp.where(sbits >= cand, 1, 0))
                return jnp.where(cnt >= _PRE_N, cand, tbits)

            tbits = jax.lax.fori_loop(0, 31, tstep, jnp.int32(0))
            sbits = jax.lax.bitcast_convert_type(score_ref[n], jnp.int32)
            n_above = jnp.sum(jnp.where(sbits > tbits, 1, 0))
            k_ties = _PRE_N - n_above
            eq = sbits == tbits

            def xstep(j, x):
                candx = x + (jnp.int32(1) << (15 - j))
                cnt = jnp.sum(
                    jnp.where(jnp.logical_and(eq, flat < candx), 1, 0))
                return jnp.where(cnt < k_ties, candx, x)

            xcut = jax.lax.fori_loop(0, 16, xstep, jnp.int32(0))
            act = jnp.logical_or(sbits > tbits,
                                 jnp.logical_and(eq, flat <= xcut))
            score_ref[n] = jnp.where(act, score_ref[n], -1.0)

        oiota = jax.lax.broadcasted_iota(jnp.int32, (1, 6), 1)
        flat_f = flat.astype(jnp.float32)
        big_f = np.float32(1 << 30)

        def vmax11(x):
            return jnp.max(jnp.max(x, axis=0, keepdims=True),
                           axis=1, keepdims=True)

        def vmin11(x):
            return jnp.min(jnp.min(x, axis=0, keepdims=True),
                           axis=1, keepdims=True)

        def vsum11(x):
            return jnp.sum(jnp.sum(x, axis=0, keepdims=True),
                           axis=1, keepdims=True)

        init = []
        for n in range(n_img):
            init.append(vmax11(score_ref[n]))
            init.append(jnp.zeros((1, 6), jnp.float32))

        def body_fn(j, carry):
            new_carry = []
            for n in range(n_img):
                m = carry[2 * n]
                lastrow = carry[2 * n + 1]
                s = score_ref[n]
                d = jnp.where(s == m, flat_f, big_f)
                idxv = vmin11(d)
                selm = jnp.where(d == idxv, 1.0, 0.0)
                g = [vsum11(src_ref[n, k] * selm)
                     for k in (0, 1, 2, 3, 4, 10, 11)]
                xci, yci, wi, hi, thi, bwi, bhi = g
                x1i = xci - bwi / 2
                y1i = yci - bhi / 2
                x2i = xci + bwi / 2
                y2i = yci + bhi / 2
                ai = (x2i - x1i) * (y2i - y1i)
                arow = jnp.zeros((1, 6), jnp.float32)
                vals6 = [xci, yci, wi, hi, thi, m]
                for k in range(6):
                    arow = jnp.where(oiota == k, vals6[k], arow)
                row = jnp.where(m > -0.5, arow, lastrow)
                out_ref[n, pl.ds(j, 1), :] = row
                xx1 = jnp.maximum(src_ref[n, 5], x1i)
                yy1 = jnp.maximum(src_ref[n, 6], y1i)
                xx2 = jnp.minimum(src_ref[n, 7], x2i)
                yy2 = jnp.minimum(src_ref[n, 8], y2i)
                iw = jnp.maximum(xx2 - xx1, 0.0)
                ih = jnp.maximum(yy2 - yy1, 0.0)
                inter = iw * ih
                iou = inter / (ai + src_ref[n, 9] - inter + 1e-9)
                news = jnp.where(iou > _THRESH, -1.0, s)
                score_ref[n] = news
                new_carry.append(vmax11(news))
                new_carry.append(row)
            return tuple(new_carry)

        jax.lax.fori_loop(0, _POST_N, body_fn, tuple(init))

    return body


def kernel(objectness, box_regression, anchors_rrects):
    N, A, H, W = objectness.shape
    nA = A * H * W
    rows = ((nA + _LANES - 1) // _LANES + 7) // 8 * 8
    padded = rows * _LANES
    pad = padded - nA

    obj = objectness.reshape(N, A, 1, H, W).transpose(0, 3, 4, 1, 2)
    obj = obj.reshape(N, nA)
    breg = box_regression.reshape(N, A, _C, H, W).transpose(0, 3, 4, 1, 2)
    breg = breg.reshape(N, nA, _C)

    obj_p = jnp.pad(obj, ((0, 0), (0, pad))).reshape(N, rows, _LANES)
    breg_t = jnp.pad(breg.transpose(0, 2, 1),
                     ((0, 0), (0, 0), (0, pad))).reshape(N, _C, rows, _LANES)
    anch_t = jnp.pad(anchors_rrects.transpose(0, 2, 1),
                     ((0, 0), (0, 0), (0, pad))).reshape(N, _C, rows, _LANES)

    out = pl.pallas_call(
        _make_body(nA, rows, N),
        grid=(1,),
        in_specs=[
            pl.BlockSpec((N, rows, _LANES), lambda i: (0, 0, 0)),
            pl.BlockSpec((N, _C, rows, _LANES), lambda i: (0, 0, 0, 0)),
            pl.BlockSpec((N, _C, rows, _LANES), lambda i: (0, 0, 0, 0)),
        ],
        out_specs=pl.BlockSpec((N, _POST_N, 6), lambda i: (0, 0, 0)),
        out_shape=jax.ShapeDtypeStruct((N, _POST_N, 6), jnp.float32),
        scratch_shapes=[
            pltpu.VMEM((N, rows, _LANES), jnp.float32),
            pltpu.VMEM((N, 12, rows, _LANES), jnp.float32),
        ],
    )(obj_p, breg_t, anch_t)
    return out
